# baseline (speedup 1.0000x reference)
import functools

import jax
import jax.numpy as jnp
from jax import lax
from jax.experimental import pallas as pl
from jax.experimental.pallas import tpu as pltpu
from jax.experimental.pallas import tpu_sc as plsc

B = 16384
D = 1024
DW = D // 4
NW = 32
ROWS_PER_W = B // NW
RBLK = 32
NBLK = ROWS_PER_W // RBLK


def _sc_permute(xf, mwf, perm):
    mesh = plsc.VectorSubcoreMesh(core_axis_name="c", subcore_axis_name="s")

    @functools.partial(
        pl.kernel,
        mesh=mesh,
        compiler_params=pltpu.CompilerParams(needs_layout_passes=False),
        out_type=[
            jax.ShapeDtypeStruct((B * D,), jnp.float32),
            jax.ShapeDtypeStruct((B * DW,), jnp.int32),
        ],
        scratch_types=[
            pltpu.VMEM((D,), jnp.int32),
            pltpu.VMEM((D,), jnp.int32),
            pltpu.VMEM((D,), jnp.int32),
            pltpu.VMEM((RBLK * D,), jnp.float32),
            pltpu.VMEM((RBLK * D,), jnp.float32),
            pltpu.VMEM((RBLK * DW,), jnp.int32),
            pltpu.VMEM((RBLK * DW,), jnp.int32),
        ],
    )
    def k(x_hbm, mw_hbm, perm_hbm, xo_hbm, mo_hbm,
          perm_v, widx_v, shf_v, xin_v, xout_v, min_v, mout_v):
        wid = lax.axis_index("s") * 2 + lax.axis_index("c")
        pltpu.sync_copy(perm_hbm, perm_v)

        iota4 = lax.iota(jnp.int32, 16) * 4

        def pre_body(c, _):
            base = 64 * (c // 4) + (c % 4)
            pv = plsc.load_gather(perm_v, [iota4 + base])
            widx_v[pl.ds(c * 16, 16)] = pv >> 2
            shf_v[pl.ds(c * 16, 16)] = (pv & 3) << 3
            return 0

        lax.fori_loop(0, 64, pre_body, 0, unroll=False)

        def blk_body(bi, _):
            r0 = wid * ROWS_PER_W + bi * RBLK
            pltpu.sync_copy(x_hbm.at[pl.ds(r0 * D, RBLK * D)], xin_v)
            pltpu.sync_copy(mw_hbm.at[pl.ds(r0 * DW, RBLK * DW)], min_v)

            def x_outer(j, _):
                colv = perm_v[pl.ds(j * 16, 16)]
                obase = j * 16

                def x_inner(rr, _):
                    for u in range(4):
                        r = rr * 4 + u
                        v = plsc.load_gather(xin_v, [colv + r * D])
                        xout_v[pl.ds(r * D + obase, 16)] = v
                    return 0

                lax.fori_loop(0, RBLK // 4, x_inner, 0, unroll=False)
                return 0

            lax.fori_loop(0, D // 16, x_outer, 0, unroll=False)

            def m_outer(j, _):
                wi = [widx_v[pl.ds((j * 4 + kk) * 16, 16)] for kk in range(4)]
                sh = [shf_v[pl.ds((j * 4 + kk) * 16, 16)] for kk in range(4)]
                obase = j * 16

                def m_inner(rr, _):
                    for u in range(2):
                        r = rr * 2 + u
                        rbase = r * DW
                        word = None
                        for kk in range(4):
                            w = plsc.load_gather(min_v, [wi[kk] + rbase])
                            a = lax.shift_right_logical(w, sh[kk]) & 1
                            word = a if kk == 0 else word | (a << (8 * kk))
                        mout_v[pl.ds(rbase + obase, 16)] = word
                    return 0

                lax.fori_loop(0, RBLK // 2, m_inner, 0, unroll=False)
                return 0

            lax.fori_loop(0, DW // 16, m_outer, 0, unroll=False)

            pltpu.sync_copy(xout_v, xo_hbm.at[pl.ds(r0 * D, RBLK * D)])
            pltpu.sync_copy(mout_v, mo_hbm.at[pl.ds(r0 * DW, RBLK * DW)])
            return 0

        lax.fori_loop(0, NBLK, blk_body, 0, unroll=False)

    return k(xf, mwf, perm)


def kernel(x, observed_mask, perm, inv_perm):
    del inv_perm
    xf = x.reshape(-1)
    mw = lax.bitcast_convert_type(
        observed_mask.astype(jnp.uint8).reshape(B, DW, 4), jnp.int32
    ).reshape(-1)
    xo_f, mo_f = _sc_permute(xf, mw, perm)
    xo = xo_f.reshape(B, D)
    mo = lax.bitcast_convert_type(
        mo_f.reshape(B, DW), jnp.uint8
    ).reshape(B, D) != 0
    return (xo, mo)


# speedup vs baseline: 2.4330x; 2.4330x over previous
import functools

import jax
import jax.numpy as jnp
from jax import lax
from jax.experimental import pallas as pl
from jax.experimental.pallas import tpu as pltpu
from jax.experimental.pallas import tpu_sc as plsc

B = 16384
D = 1024
NW = 32
ROWS_PER_W = B // NW
RBLK = 32
NBLK = ROWS_PER_W // RBLK


def _sc_permute_x(xf, perm):
    mesh = plsc.VectorSubcoreMesh(core_axis_name="c", subcore_axis_name="s")

    @functools.partial(
        pl.kernel,
        mesh=mesh,
        compiler_params=pltpu.CompilerParams(needs_layout_passes=False),
        out_type=jax.ShapeDtypeStruct((B * D,), jnp.float32),
        scratch_types=[
            pltpu.VMEM((D,), jnp.int32),
            pltpu.VMEM((RBLK * D,), jnp.float32),
            pltpu.VMEM((RBLK * D,), jnp.float32),
        ],
    )
    def k(x_hbm, perm_hbm, xo_hbm, perm_v, xin_v, xout_v):
        wid = lax.axis_index("s") * 2 + lax.axis_index("c")
        pltpu.sync_copy(perm_hbm, perm_v)

        def blk_body(bi, _):
            r0 = wid * ROWS_PER_W + bi * RBLK
            pltpu.sync_copy(x_hbm.at[pl.ds(r0 * D, RBLK * D)], xin_v)

            def x_outer(j, _):
                colv = perm_v[pl.ds(j * 16, 16)]
                obase = j * 16
                for r in range(RBLK):
                    v = plsc.load_gather(xin_v, [colv + r * D])
                    xout_v[pl.ds(r * D + obase, 16)] = v
                return 0

            lax.fori_loop(0, D // 16, x_outer, 0, unroll=False)

            pltpu.sync_copy(xout_v, xo_hbm.at[pl.ds(r0 * D, RBLK * D)])
            return 0

        lax.fori_loop(0, NBLK, blk_body, 0, unroll=False)

    return k(xf, perm)


def _tc_build_p(perm):
    def build(perm_ref, p_ref):
        col = lax.broadcasted_iota(jnp.int32, (D, D), 0)
        pj = perm_ref[...]
        p_ref[...] = (pj[None, :] == col).astype(jnp.bfloat16)

    return pl.pallas_call(
        build,
        out_shape=jax.ShapeDtypeStruct((D, D), jnp.bfloat16),
    )(perm)


def _tc_permute_mask(mask, p_mat):
    rb = 1024

    def mm(m_ref, p_ref, o_ref):
        mb = m_ref[...].astype(jnp.bfloat16)
        acc = jnp.dot(mb, p_ref[...], preferred_element_type=jnp.float32)
        o_ref[...] = acc > 0.5

    return pl.pallas_call(
        mm,
        grid=(B // rb,),
        in_specs=[
            pl.BlockSpec((rb, D), lambda i: (i, 0)),
            pl.BlockSpec((D, D), lambda i: (0, 0)),
        ],
        out_specs=pl.BlockSpec((rb, D), lambda i: (i, 0)),
        out_shape=jax.ShapeDtypeStruct((B, D), jnp.bool_),
    )(mask, p_mat)


def kernel(x, observed_mask, perm, inv_perm):
    del inv_perm
    xo_f = _sc_permute_x(x.reshape(-1), perm)
    p_mat = _tc_build_p(perm)
    mo = _tc_permute_mask(observed_mask, p_mat)
    return (xo_f.reshape(B, D), mo)


# speedup vs baseline: 5.0129x; 2.0603x over previous
import functools

import jax
import jax.numpy as jnp
from jax import lax
from jax.experimental import pallas as pl
from jax.experimental.pallas import tpu as pltpu
from jax.experimental.pallas import tpu_sc as plsc

B = 16384
D = 1024
NW = 32
ROWS_PER_W = B // NW
RBLK = 16
NBLK = ROWS_PER_W // RBLK


def _sc_permute_x(x, perm):
    mesh = plsc.VectorSubcoreMesh(core_axis_name="c", subcore_axis_name="s")

    @functools.partial(
        pl.kernel,
        mesh=mesh,
        compiler_params=pltpu.CompilerParams(needs_layout_passes=False),
        out_type=jax.ShapeDtypeStruct((B, D), jnp.float32),
        scratch_types=[
            pltpu.VMEM((D,), jnp.int32),
            pltpu.VMEM((RBLK, D), jnp.float32),
            pltpu.VMEM((RBLK, D), jnp.float32),
            pltpu.VMEM((RBLK, D), jnp.float32),
            pltpu.VMEM((RBLK, D), jnp.float32),
            pltpu.SemaphoreType.DMA,
            pltpu.SemaphoreType.DMA,
            pltpu.SemaphoreType.DMA,
            pltpu.SemaphoreType.DMA,
        ],
    )
    def k(x_hbm, perm_hbm, xo_hbm,
          perm_v, xin0, xin1, xout0, xout1, si0, si1, so0, so1):
        wid = lax.axis_index("s") * 2 + lax.axis_index("c")
        base = wid * ROWS_PER_W
        pltpu.sync_copy(perm_hbm, perm_v)

        xin = (xin0, xin1)
        xout = (xout0, xout1)
        si = (si0, si1)
        so = (so0, so1)
        rsp = [jnp.full((16,), r, jnp.int32) for r in range(RBLK)]

        def start_in(bi, p):
            pltpu.make_async_copy(
                x_hbm.at[pl.ds(base + bi * RBLK, RBLK)], xin[p], si[p]
            ).start()

        def wait_in(p):
            pltpu.make_async_copy(
                x_hbm.at[pl.ds(base, RBLK)], xin[p], si[p]
            ).wait()

        def start_out(bi, p):
            pltpu.make_async_copy(
                xout[p], xo_hbm.at[pl.ds(base + bi * RBLK, RBLK)], so[p]
            ).start()

        def wait_out(p):
            pltpu.make_async_copy(
                xout[p], xo_hbm.at[pl.ds(base, RBLK)], so[p]
            ).wait()

        def compute(p):
            xin_p = xin[p]
            xout_p = xout[p]

            def x_outer(j, _):
                obase = j * 16
                colv = perm_v[pl.ds(obase, 16)]
                for g in range(0, RBLK, 8):
                    vals = [
                        plsc.load_gather(xin_p, [rsp[g + u], colv])
                        for u in range(8)
                    ]
                    for u in range(8):
                        xout_p[g + u, pl.ds(obase, 16)] = vals[u]
                return 0

            lax.fori_loop(0, D // 16, x_outer, 0, unroll=False)

        start_in(0, 0)

        def body(hi, _):
            for p in (0, 1):
                bi = hi * 2 + p

                @pl.when(bi + 1 < NBLK)
                def _():
                    start_in(bi + 1, 1 - p)

                wait_in(p)

                @pl.when(bi >= 2)
                def _():
                    wait_out(p)

                compute(p)
                start_out(bi, p)
            return 0

        lax.fori_loop(0, NBLK // 2, body, 0, unroll=False)
        wait_out(0)
        wait_out(1)

    return k(x, perm)


def _tc_build_p(perm):
    def build(perm_ref, p_ref):
        col = lax.broadcasted_iota(jnp.int32, (D, D), 0)
        pj = perm_ref[...]
        p_ref[...] = (pj[None, :] == col).astype(jnp.bfloat16)

    return pl.pallas_call(
        build,
        out_shape=jax.ShapeDtypeStruct((D, D), jnp.bfloat16),
    )(perm)


def _tc_permute_mask(mask, p_mat):
    rb = 1024

    def mm(m_ref, p_ref, o_ref):
        mb = m_ref[...].astype(jnp.bfloat16)
        acc = jnp.dot(mb, p_ref[...], preferred_element_type=jnp.float32)
        o_ref[...] = acc > 0.5

    return pl.pallas_call(
        mm,
        grid=(B // rb,),
        in_specs=[
            pl.BlockSpec((rb, D), lambda i: (i, 0)),
            pl.BlockSpec((D, D), lambda i: (0, 0)),
        ],
        out_specs=pl.BlockSpec((rb, D), lambda i: (i, 0)),
        out_shape=jax.ShapeDtypeStruct((B, D), jnp.bool_),
    )(mask, p_mat)


def kernel(x, observed_mask, perm, inv_perm):
    del inv_perm
    xo = _sc_permute_x(x, perm)
    p_mat = _tc_build_p(perm)
    mo = _tc_permute_mask(observed_mask, p_mat)
    return (xo, mo)


# speedup vs baseline: 6.9813x; 1.3927x over previous
import functools

import jax
import jax.numpy as jnp
from jax import lax
from jax.experimental import pallas as pl
from jax.experimental.pallas import tpu as pltpu
from jax.experimental.pallas import tpu_sc as plsc

B = 16384
D = 1024
NW = 32
ROWS_PER_W = B // NW
RBLK = 16
NBLK = ROWS_PER_W // RBLK


def _sc_permute_x(x, perm):
    mesh = plsc.VectorSubcoreMesh(core_axis_name="c", subcore_axis_name="s")

    @functools.partial(
        pl.kernel,
        mesh=mesh,
        compiler_params=pltpu.CompilerParams(needs_layout_passes=False),
        out_type=jax.ShapeDtypeStruct((B, D), jnp.float32),
        scratch_types=[
            pltpu.VMEM((D,), jnp.int32),
            pltpu.VMEM((RBLK, D), jnp.float32),
            pltpu.VMEM((RBLK, D), jnp.float32),
            pltpu.VMEM((RBLK, D), jnp.float32),
            pltpu.VMEM((RBLK, D), jnp.float32),
            pltpu.SemaphoreType.DMA,
            pltpu.SemaphoreType.DMA,
            pltpu.SemaphoreType.DMA,
            pltpu.SemaphoreType.DMA,
        ],
    )
    def k(x_hbm, perm_hbm, xo_hbm,
          perm_v, xin0, xin1, xout0, xout1, si0, si1, so0, so1):
        wid = lax.axis_index("s") * 2 + lax.axis_index("c")
        base = wid * ROWS_PER_W
        pltpu.sync_copy(perm_hbm, perm_v)

        xin = (xin0, xin1)
        xout = (xout0, xout1)
        si = (si0, si1)
        so = (so0, so1)
        rsp = [jnp.full((16,), r, jnp.int32) for r in range(RBLK)]

        def start_in(bi, p):
            pltpu.make_async_copy(
                x_hbm.at[pl.ds(base + bi * RBLK, RBLK)], xin[p], si[p]
            ).start()

        def wait_in(p):
            pltpu.make_async_copy(
                x_hbm.at[pl.ds(base, RBLK)], xin[p], si[p]
            ).wait()

        def start_out(bi, p):
            pltpu.make_async_copy(
                xout[p], xo_hbm.at[pl.ds(base + bi * RBLK, RBLK)], so[p]
            ).start()

        def wait_out(p):
            pltpu.make_async_copy(
                xout[p], xo_hbm.at[pl.ds(base, RBLK)], so[p]
            ).wait()

        def compute(p):
            xin_p = xin[p]
            xout_p = xout[p]

            def x_outer(j, _):
                obase = j * 16
                colv = perm_v[pl.ds(obase, 16)]
                for g in range(0, RBLK, 8):
                    vals = [
                        plsc.load_gather(xin_p, [rsp[g + u], colv])
                        for u in range(8)
                    ]
                    for u in range(8):
                        xout_p[g + u, pl.ds(obase, 16)] = vals[u]
                return 0

            lax.fori_loop(0, D // 16, x_outer, 0, unroll=False)

        start_in(0, 0)

        def body(hi, _):
            for p in (0, 1):
                bi = hi * 2 + p

                @pl.when(bi + 1 < NBLK)
                def _():
                    start_in(bi + 1, 1 - p)

                wait_in(p)

                @pl.when(bi >= 2)
                def _():
                    wait_out(p)

                compute(p)
                start_out(bi, p)
            return 0

        lax.fori_loop(0, NBLK // 2, body, 0, unroll=False)
        wait_out(0)
        wait_out(1)

    return k(x, perm)


def _tc_build_p(perm):
    def build(perm_ref, p_ref):
        col = lax.broadcasted_iota(jnp.int32, (D, D), 0)
        pj = perm_ref[...]
        p_ref[...] = (pj[None, :] == col).astype(jnp.bfloat16)

    return pl.pallas_call(
        build,
        out_shape=jax.ShapeDtypeStruct((D, D), jnp.bfloat16),
    )(perm)


def _tc_permute_mask(mask_bf, p_mat):
    rb = 4096

    def mm(m_ref, p_ref, o_ref):
        acc = jnp.dot(m_ref[...], p_ref[...],
                      preferred_element_type=jnp.float32)
        o_ref[...] = acc.astype(jnp.bfloat16)

    return pl.pallas_call(
        mm,
        grid=(B // rb,),
        in_specs=[
            pl.BlockSpec((rb, D), lambda i: (i, 0)),
            pl.BlockSpec((D, D), lambda i: (0, 0)),
        ],
        out_specs=pl.BlockSpec((rb, D), lambda i: (i, 0)),
        out_shape=jax.ShapeDtypeStruct((B, D), jnp.bfloat16),
    )(mask_bf, p_mat)


def kernel(x, observed_mask, perm, inv_perm):
    del inv_perm
    xo = _sc_permute_x(x, perm)
    p_mat = _tc_build_p(perm)
    mo_bf = _tc_permute_mask(observed_mask.astype(jnp.bfloat16), p_mat)
    return (xo, mo_bf != 0)


# speedup vs baseline: 7.8336x; 1.1221x over previous
import functools

import jax
import jax.numpy as jnp
from jax import lax
from jax.experimental import pallas as pl
from jax.experimental.pallas import tpu as pltpu
from jax.experimental.pallas import tpu_sc as plsc

B = 16384
D = 1024
NW = 32
ROWS_PER_W = B // NW
RBLK = 16
NBLK = ROWS_PER_W // RBLK


def _sc_permute_x(x, perm):
    mesh = plsc.VectorSubcoreMesh(core_axis_name="c", subcore_axis_name="s")

    @functools.partial(
        pl.kernel,
        mesh=mesh,
        compiler_params=pltpu.CompilerParams(needs_layout_passes=False),
        out_type=jax.ShapeDtypeStruct((B, D), jnp.float32),
        scratch_types=[
            pltpu.VMEM((D,), jnp.int32),
            pltpu.VMEM((RBLK, D), jnp.float32),
            pltpu.VMEM((RBLK, D), jnp.float32),
            pltpu.VMEM((RBLK, D), jnp.float32),
            pltpu.VMEM((RBLK, D), jnp.float32),
            pltpu.SemaphoreType.DMA,
            pltpu.SemaphoreType.DMA,
            pltpu.SemaphoreType.DMA,
            pltpu.SemaphoreType.DMA,
        ],
    )
    def k(x_hbm, perm_hbm, xo_hbm,
          perm_v, xin0, xin1, xout0, xout1, si0, si1, so0, so1):
        wid = lax.axis_index("s") * 2 + lax.axis_index("c")
        base = wid * ROWS_PER_W
        pltpu.sync_copy(perm_hbm, perm_v)

        xin = (xin0, xin1)
        xout = (xout0, xout1)
        si = (si0, si1)
        so = (so0, so1)
        rsp = [jnp.full((16,), r, jnp.int32) for r in range(RBLK)]

        def start_in(bi, p):
            pltpu.make_async_copy(
                x_hbm.at[pl.ds(base + bi * RBLK, RBLK)], xin[p], si[p]
            ).start()

        def wait_in(p):
            pltpu.make_async_copy(
                x_hbm.at[pl.ds(base, RBLK)], xin[p], si[p]
            ).wait()

        def start_out(bi, p):
            pltpu.make_async_copy(
                xout[p], xo_hbm.at[pl.ds(base + bi * RBLK, RBLK)], so[p]
            ).start()

        def wait_out(p):
            pltpu.make_async_copy(
                xout[p], xo_hbm.at[pl.ds(base, RBLK)], so[p]
            ).wait()

        def compute(p):
            xin_p = xin[p]
            xout_p = xout[p]

            def x_outer(j, _):
                obase = j * 16
                colv = perm_v[pl.ds(obase, 16)]
                for g in range(0, RBLK, 8):
                    vals = [
                        plsc.load_gather(xin_p, [rsp[g + u], colv])
                        for u in range(8)
                    ]
                    for u in range(8):
                        xout_p[g + u, pl.ds(obase, 16)] = vals[u]
                return 0

            lax.fori_loop(0, D // 16, x_outer, 0, unroll=False)

        start_in(0, 0)

        def body(hi, _):
            for p in (0, 1):
                bi = hi * 2 + p

                @pl.when(bi + 1 < NBLK)
                def _():
                    start_in(bi + 1, 1 - p)

                wait_in(p)

                @pl.when(bi >= 2)
                def _():
                    wait_out(p)

                compute(p)
                start_out(bi, p)
            return 0

        lax.fori_loop(0, NBLK // 2, body, 0, unroll=False)
        wait_out(0)
        wait_out(1)

    return k(x, perm)


def _tc_build_p(perm):
    def build(perm_ref, p_ref):
        col = lax.broadcasted_iota(jnp.int32, (D, D), 0)
        pj = perm_ref[...]
        p_ref[...] = (pj[None, :] == col).astype(jnp.int8)

    return pl.pallas_call(
        build,
        out_shape=jax.ShapeDtypeStruct((D, D), jnp.int8),
    )(perm)


def _tc_permute_mask(mask_i8, p_mat):
    rb = 4096

    def mm(m_ref, p_ref, o_ref):
        acc = jnp.dot(m_ref[...], p_ref[...],
                      preferred_element_type=jnp.int32)
        o_ref[...] = acc.astype(jnp.int8)

    return pl.pallas_call(
        mm,
        grid=(B // rb,),
        in_specs=[
            pl.BlockSpec((rb, D), lambda i: (i, 0)),
            pl.BlockSpec((D, D), lambda i: (0, 0)),
        ],
        out_specs=pl.BlockSpec((rb, D), lambda i: (i, 0)),
        out_shape=jax.ShapeDtypeStruct((B, D), jnp.int8),
    )(mask_i8, p_mat)


def kernel(x, observed_mask, perm, inv_perm):
    del inv_perm
    xo = _sc_permute_x(x, perm)
    p_mat = _tc_build_p(perm)
    mo_i8 = _tc_permute_mask(observed_mask.astype(jnp.int8), p_mat)
    return (xo, mo_i8 != 0)


# speedup vs baseline: 8.2001x; 1.0468x over previous
import functools

import jax
import jax.numpy as jnp
from jax import lax
from jax.experimental import pallas as pl
from jax.experimental.pallas import tpu as pltpu
from jax.experimental.pallas import tpu_sc as plsc

B = 16384
D = 1024
NW = 32
ROWS_PER_W = B // NW
RBLK = 16
NBLK = ROWS_PER_W // RBLK


def _sc_permute_x(x, perm):
    mesh = plsc.VectorSubcoreMesh(core_axis_name="c", subcore_axis_name="s")

    @functools.partial(
        pl.kernel,
        mesh=mesh,
        compiler_params=pltpu.CompilerParams(needs_layout_passes=False),
        out_type=jax.ShapeDtypeStruct((B, D), jnp.float32),
        scratch_types=[
            pltpu.VMEM((D,), jnp.int32),
            pltpu.VMEM((RBLK, D), jnp.float32),
            pltpu.VMEM((RBLK, D), jnp.float32),
            pltpu.VMEM((RBLK, D), jnp.float32),
            pltpu.VMEM((RBLK, D), jnp.float32),
            pltpu.SemaphoreType.DMA,
            pltpu.SemaphoreType.DMA,
            pltpu.SemaphoreType.DMA,
            pltpu.SemaphoreType.DMA,
        ],
    )
    def k(x_hbm, perm_hbm, xo_hbm,
          perm_v, xin0, xin1, xout0, xout1, si0, si1, so0, so1):
        wid = lax.axis_index("s") * 2 + lax.axis_index("c")
        base = wid * ROWS_PER_W
        pltpu.sync_copy(perm_hbm, perm_v)

        xin = (xin0, xin1)
        xout = (xout0, xout1)
        si = (si0, si1)
        so = (so0, so1)
        rsp = [jnp.full((16,), r, jnp.int32) for r in range(RBLK)]

        def start_in(bi, p):
            pltpu.make_async_copy(
                x_hbm.at[pl.ds(base + bi * RBLK, RBLK)], xin[p], si[p]
            ).start()

        def wait_in(p):
            pltpu.make_async_copy(
                x_hbm.at[pl.ds(base, RBLK)], xin[p], si[p]
            ).wait()

        def start_out(bi, p):
            pltpu.make_async_copy(
                xout[p], xo_hbm.at[pl.ds(base + bi * RBLK, RBLK)], so[p]
            ).start()

        def wait_out(p):
            pltpu.make_async_copy(
                xout[p], xo_hbm.at[pl.ds(base, RBLK)], so[p]
            ).wait()

        def compute(p):
            xin_p = xin[p]
            xout_p = xout[p]
            H = RBLK // 2

            def gather_half(colv, g0):
                return tuple(
                    plsc.load_gather(xin_p, [rsp[g0 + u], colv])
                    for u in range(H)
                )

            def store_half(vals, obase, g0):
                for u in range(H):
                    xout_p[g0 + u, pl.ds(obase, 16)] = vals[u]

            colv0 = perm_v[pl.ds(0, 16)]
            lo0 = gather_half(colv0, 0)
            store_half(lo0, 0, 0)
            hi0 = gather_half(colv0, H)

            def x_outer(j, carry):
                obase = j * 16
                colv = perm_v[pl.ds(obase, 16)]
                lo = gather_half(colv, 0)
                store_half(carry, obase - 16, H)
                hi = gather_half(colv, H)
                store_half(lo, obase, 0)
                return hi

            last = lax.fori_loop(1, D // 16, x_outer, hi0, unroll=False)
            store_half(last, D - 16, H)

        start_in(0, 0)

        def body(hi, _):
            for p in (0, 1):
                bi = hi * 2 + p

                @pl.when(bi + 1 < NBLK)
                def _():
                    start_in(bi + 1, 1 - p)

                wait_in(p)

                @pl.when(bi >= 2)
                def _():
                    wait_out(p)

                compute(p)
                start_out(bi, p)
            return 0

        lax.fori_loop(0, NBLK // 2, body, 0, unroll=False)
        wait_out(0)
        wait_out(1)

    return k(x, perm)


def _tc_build_p(perm):
    def build(perm_ref, p_ref):
        col = lax.broadcasted_iota(jnp.int32, (D, D), 0)
        pj = perm_ref[...]
        p_ref[...] = (pj[None, :] == col).astype(jnp.int8)

    return pl.pallas_call(
        build,
        out_shape=jax.ShapeDtypeStruct((D, D), jnp.int8),
    )(perm)


def _tc_permute_mask(mask_i8, p_mat):
    rb = 4096

    def mm(m_ref, p_ref, o_ref):
        acc = jnp.dot(m_ref[...], p_ref[...],
                      preferred_element_type=jnp.int32)
        o_ref[...] = acc.astype(jnp.int8)

    return pl.pallas_call(
        mm,
        grid=(B // rb,),
        in_specs=[
            pl.BlockSpec((rb, D), lambda i: (i, 0)),
            pl.BlockSpec((D, D), lambda i: (0, 0)),
        ],
        out_specs=pl.BlockSpec((rb, D), lambda i: (i, 0)),
        out_shape=jax.ShapeDtypeStruct((B, D), jnp.int8),
    )(mask_i8, p_mat)


def kernel(x, observed_mask, perm, inv_perm):
    del inv_perm
    xo = _sc_permute_x(x, perm)
    p_mat = _tc_build_p(perm)
    mo_i8 = _tc_permute_mask(observed_mask.astype(jnp.int8), p_mat)
    return (xo, mo_i8 != 0)


# speedup vs baseline: 8.2959x; 1.0117x over previous
import functools

import jax
import jax.numpy as jnp
from jax import lax
from jax.experimental import pallas as pl
from jax.experimental.pallas import tpu as pltpu
from jax.experimental.pallas import tpu_sc as plsc

B = 16384
D = 1024
NW = 32
ROWS_PER_W = B // NW
RBLK = 16
NBLK = ROWS_PER_W // RBLK


def _sc_permute_x(x, perm):
    mesh = plsc.VectorSubcoreMesh(core_axis_name="c", subcore_axis_name="s")

    @functools.partial(
        pl.kernel,
        mesh=mesh,
        compiler_params=pltpu.CompilerParams(needs_layout_passes=False),
        out_type=jax.ShapeDtypeStruct((B, D), jnp.float32),
        scratch_types=[
            pltpu.VMEM((D,), jnp.int32),
            pltpu.VMEM((RBLK, D), jnp.float32),
            pltpu.VMEM((RBLK, D), jnp.float32),
            pltpu.VMEM((RBLK, D), jnp.float32),
            pltpu.VMEM((RBLK, D), jnp.float32),
            pltpu.SemaphoreType.DMA,
            pltpu.SemaphoreType.DMA,
            pltpu.SemaphoreType.DMA,
            pltpu.SemaphoreType.DMA,
        ],
    )
    def k(x_hbm, perm_hbm, xo_hbm,
          perm_v, xin0, xin1, xout0, xout1, si0, si1, so0, so1):
        wid = lax.axis_index("s") * 2 + lax.axis_index("c")
        base = wid * ROWS_PER_W

        xin = (xin0, xin1)
        xout = (xout0, xout1)
        si = (si0, si1)
        so = (so0, so1)
        rsp = [jnp.full((16,), r, jnp.int32) for r in range(RBLK)]

        def start_in(bi, p):
            pltpu.make_async_copy(
                x_hbm.at[pl.ds(base + bi * RBLK, RBLK)], xin[p], si[p]
            ).start()

        def wait_in(p):
            pltpu.make_async_copy(
                x_hbm.at[pl.ds(base, RBLK)], xin[p], si[p]
            ).wait()

        def start_out(bi, p):
            pltpu.make_async_copy(
                xout[p], xo_hbm.at[pl.ds(base + bi * RBLK, RBLK)], so[p]
            ).start()

        def wait_out(p):
            pltpu.make_async_copy(
                xout[p], xo_hbm.at[pl.ds(base, RBLK)], so[p]
            ).wait()

        def compute(p):
            xin_p = xin[p]
            xout_p = xout[p]
            H = RBLK // 2

            def gather_half(colv, g0):
                return tuple(
                    plsc.load_gather(xin_p, [rsp[g0 + u], colv])
                    for u in range(H)
                )

            def store_half(vals, obase, g0):
                for u in range(H):
                    xout_p[g0 + u, pl.ds(obase, 16)] = vals[u]

            colv0 = perm_v[pl.ds(0, 16)]
            lo0 = gather_half(colv0, 0)
            store_half(lo0, 0, 0)
            hi0 = gather_half(colv0, H)

            def x_outer(j, carry):
                obase = j * 16
                colv = perm_v[pl.ds(obase, 16)]
                lo = gather_half(colv, 0)
                store_half(carry, obase - 16, H)
                hi = gather_half(colv, H)
                store_half(lo, obase, 0)
                return hi

            last = lax.fori_loop(1, D // 16, x_outer, hi0, unroll=False)
            store_half(last, D - 16, H)

        start_in(0, 0)
        start_in(1, 1)
        pltpu.sync_copy(perm_hbm, perm_v)

        def body(hi, _):
            for p in (0, 1):
                bi = hi * 2 + p

                wait_in(p)

                @pl.when(bi >= 2)
                def _():
                    wait_out(p)

                compute(p)

                @pl.when(bi + 2 < NBLK)
                def _():
                    start_in(bi + 2, p)

                start_out(bi, p)
            return 0

        lax.fori_loop(0, NBLK // 2, body, 0, unroll=False)
        wait_out(0)
        wait_out(1)

    return k(x, perm)


def _tc_build_p(perm):
    def build(perm_ref, p_ref):
        col = lax.broadcasted_iota(jnp.int32, (D, D), 0)
        pj = perm_ref[...]
        p_ref[...] = (pj[None, :] == col).astype(jnp.int8)

    return pl.pallas_call(
        build,
        out_shape=jax.ShapeDtypeStruct((D, D), jnp.int8),
    )(perm)


def _tc_permute_mask(mask_i8, p_mat):
    rb = 2048

    def mm(m_ref, p_ref, o_ref):
        acc = jnp.dot(m_ref[...], p_ref[...],
                      preferred_element_type=jnp.int32)
        o_ref[...] = acc.astype(jnp.int8)

    return pl.pallas_call(
        mm,
        grid=(B // rb,),
        in_specs=[
            pl.BlockSpec((rb, D), lambda i: (i, 0)),
            pl.BlockSpec((D, D), lambda i: (0, 0)),
        ],
        out_specs=pl.BlockSpec((rb, D), lambda i: (i, 0)),
        out_shape=jax.ShapeDtypeStruct((B, D), jnp.int8),
    )(mask_i8, p_mat)


def kernel(x, observed_mask, perm, inv_perm):
    del inv_perm
    xo = _sc_permute_x(x, perm)
    p_mat = _tc_build_p(perm)
    mo_i8 = _tc_permute_mask(observed_mask.astype(jnp.int8), p_mat)
    return (xo, mo_i8 != 0)
